# baseline (speedup 1.0000x reference)
import functools

import jax
import jax.numpy as jnp
from jax import lax
from jax.experimental import pallas as pl
from jax.experimental.pallas import tpu as pltpu
from jax.experimental.pallas import tpu_sc as plsc

H = 2048
W = 2048
C = 16
M = 1048576

NC = 2
NS = 16
NW = NC * NS

B = 128
CHUNK = M // NW
NBLK = CHUNK // B
GPB = B // 16

_F = jnp.float32
_I = jnp.int32


def _sc_body(table_hbm, xs_hbm, ys_hbm, out_hbm,
             xs_v, ys_v, i00_v, i01_v, i10_v, i11_v, wx_v, wy_v,
             r00_v, r01_v, r10_v, r11_v, out_v, sem):
    wid = lax.axis_index("s") * NC + lax.axis_index("c")
    chunk_base = wid * CHUNK

    def block(g, carry):
        base = chunk_base + g * B
        pltpu.sync_copy(xs_hbm.at[pl.ds(base, B)], xs_v)
        pltpu.sync_copy(ys_hbm.at[pl.ds(base, B)], ys_v)

        def pass1(i, c1):
            sl = pl.ds(i * 16, 16)
            x = xs_v[sl]
            y = ys_v[sl]
            u = 2.0 * x - 1.0
            v = 2.0 * y - 1.0
            xp = (u + 1.0) * 0.5 * (W - 1)
            yp = (v + 1.0) * 0.5 * (H - 1)
            xp = jnp.minimum(jnp.maximum(xp, 0.0), float(W - 1))
            yp = jnp.minimum(jnp.maximum(yp, 0.0), float(H - 1))
            x0i = xp.astype(_I)
            y0i = yp.astype(_I)
            x0f = x0i.astype(_F)
            y0f = y0i.astype(_F)
            x1i = jnp.minimum(x0i + 1, W - 1)
            y1i = jnp.minimum(y0i + 1, H - 1)
            wx_v[sl] = xp - x0f
            wy_v[sl] = yp - y0f
            y0w = y0i * W
            y1w = y1i * W
            i00_v[sl] = y0w + x0i
            i01_v[sl] = y0w + x1i
            i10_v[sl] = y1w + x0i
            i11_v[sl] = y1w + x1i
            return c1

        lax.fori_loop(0, GPB, pass1, 0, unroll=GPB)

        cp0 = pltpu.async_copy(table_hbm.at[i00_v], r00_v, sem)
        cp1 = pltpu.async_copy(table_hbm.at[i01_v], r01_v, sem)
        cp2 = pltpu.async_copy(table_hbm.at[i10_v], r10_v, sem)
        cp3 = pltpu.async_copy(table_hbm.at[i11_v], r11_v, sem)
        cp0.wait()
        cp1.wait()
        cp2.wait()
        cp3.wait()

        giota = lax.iota(_I, 16)

        def pass2(i, c2):
            sl = pl.ds(i * 16, 16)
            pidx = giota + i * 16
            wx = wx_v[sl]
            wy = wy_v[sl]
            w11 = wx * wy
            w10 = wy - w11
            w01 = wx - w11
            w00 = (1.0 - wx) - w10
            for c in range(C):
                cvec = jnp.full((16,), c, dtype=_I)
                ia = plsc.load_gather(r00_v, [pidx, cvec])
                ib = plsc.load_gather(r01_v, [pidx, cvec])
                ic = plsc.load_gather(r10_v, [pidx, cvec])
                id_ = plsc.load_gather(r11_v, [pidx, cvec])
                val = ia * w00 + ib * w01 + ic * w10 + id_ * w11
                plsc.store_scatter(out_v, [pidx, cvec], val)
            return c2

        lax.fori_loop(0, GPB, pass2, 0, unroll=GPB)

        pltpu.sync_copy(out_v, out_hbm.at[pl.ds(base, B)])
        return carry

    lax.fori_loop(0, NBLK, block, 0)


@jax.jit
def _grid_sample_sc(table, xs, ys):
    mesh = plsc.VectorSubcoreMesh(core_axis_name="c", subcore_axis_name="s")
    f = pl.kernel(
        _sc_body,
        out_type=jax.ShapeDtypeStruct((M, C), _F),
        mesh=mesh,
        scratch_types=[
            pltpu.VMEM((B,), _F),
            pltpu.VMEM((B,), _F),
            pltpu.VMEM((B,), _I),
            pltpu.VMEM((B,), _I),
            pltpu.VMEM((B,), _I),
            pltpu.VMEM((B,), _I),
            pltpu.VMEM((B,), _F),
            pltpu.VMEM((B,), _F),
            pltpu.VMEM((B, C), _F),
            pltpu.VMEM((B, C), _F),
            pltpu.VMEM((B, C), _F),
            pltpu.VMEM((B, C), _F),
            pltpu.VMEM((B, C), _F),
            pltpu.SemaphoreType.DMA,
        ],
        compiler_params=pltpu.CompilerParams(
            needs_layout_passes=False, use_tc_tiling_on_sc=False
        ),
    )
    return f(table, xs, ys)


def kernel(coords_local_xy, w_grid):
    coords = coords_local_xy.astype(jnp.float32)
    xs = coords[:, 0]
    ys = coords[:, 1]
    table = jnp.transpose(w_grid[0], (1, 2, 0)).reshape(H * W, C)
    return _grid_sample_sc(table, xs, ys)
